# baseline (speedup 1.0000x reference)
import functools

import jax
import jax.numpy as jnp
from jax import lax
from jax.experimental import pallas as pl
from jax.experimental.pallas import tpu as pltpu
from jax.experimental.pallas import tpu_sc as plsc

N = 4194304
K = N // 2
NC, NS, L = 2, 16, 16
NW = NC * NS
E = N // NW
CHUNK = 8192
NCHUNK = E // CHUNK
B = 4096
UNROLL = 16
RSTRIDE = B + 1
HSIZE = -(-(L * RSTRIDE) // (L * UNROLL)) * (L * UNROLL)

_mesh = plsc.VectorSubcoreMesh(core_axis_name="c", subcore_axis_name="s")


def _worker_id():
    return lax.axis_index("s") * NC + lax.axis_index("c")


def _hist_body(hv_hbm, sel_hbm, out_hbm, buf, hist, red, bvec, sem_a, sem_b,
               *, shift, filt_shift):
    wid = _worker_id()
    base = wid * E
    lane = lax.iota(jnp.int32, L)
    laneoff = lane * RSTRIDE
    ones = jnp.ones((L,), jnp.int32)
    zeros = jnp.zeros((L,), jnp.int32)

    if filt_shift is not None:
        pltpu.sync_copy(sel_hbm.at[0, pl.ds(0, L)], bvec)
        fval = bvec[...]

    @plsc.parallel_loop(0, HSIZE // L, 1, unroll=UNROLL)
    def _(i):
        hist[pl.ds(i * L, L)] = zeros

    sems = (sem_a, sem_b)
    copies = [
        pltpu.make_async_copy(
            hv_hbm.at[pl.ds(base + ch * CHUNK, CHUNK)], buf.at[ch % 2],
            sems[ch % 2])
        for ch in range(NCHUNK)
    ]
    copies[0].start()
    for ch in range(NCHUNK):
        if ch + 1 < NCHUNK:
            copies[ch + 1].start()
        copies[ch].wait()
        slot = ch % 2

        @plsc.parallel_loop(0, CHUNK // L, 1, unroll=UNROLL)
        def _(i):
            v = buf[slot, pl.ds(i * L, L)]
            bits = lax.bitcast_convert_type(v, jnp.int32)
            a = bits & jnp.int32(0x7FFFFFFF)
            bucket = lax.shift_right_logical(a, shift) & jnp.int32(B - 1)
            idx = laneoff + bucket
            if filt_shift is None:
                plsc.addupdate_scatter(hist, [idx], ones)
            else:
                m = lax.shift_right_logical(a, filt_shift) == fval
                plsc.addupdate_scatter(hist, [idx], ones, mask=m)

    @plsc.parallel_loop(0, B // L, 1, unroll=2)
    def _(j):
        acc = hist[pl.ds(j * L, L)]
        for l in range(1, L):
            acc = acc + hist[pl.ds(l * RSTRIDE + j * L, L)]
        red[pl.ds(j * L, L)] = acc
    pltpu.sync_copy(red, out_hbm.at[wid])


_hist_scratch = [
    pltpu.VMEM((2, CHUNK), jnp.float32),
    pltpu.VMEM((HSIZE,), jnp.int32),
    pltpu.VMEM((B,), jnp.int32),
    pltpu.VMEM((L,), jnp.int32),
    pltpu.SemaphoreType.DMA,
    pltpu.SemaphoreType.DMA,
]


@functools.partial(pl.kernel,
                   out_type=jax.ShapeDtypeStruct((NW, B), jnp.int32),
                   mesh=_mesh, scratch_types=_hist_scratch,
                   compiler_params=pltpu.CompilerParams(
                       needs_layout_passes=False))
def _sc_hist1(hv_hbm, out_hbm, buf, hist, red, bvec, sem_a, sem_b):
    _hist_body(hv_hbm, None, out_hbm, buf, hist, red, bvec, sem_a, sem_b,
               shift=19, filt_shift=None)


@functools.partial(pl.kernel,
                   out_type=jax.ShapeDtypeStruct((NW, B), jnp.int32),
                   mesh=_mesh, scratch_types=_hist_scratch,
                   compiler_params=pltpu.CompilerParams(
                       needs_layout_passes=False))
def _sc_hist2(hv_hbm, sel_hbm, out_hbm, buf, hist, red, bvec, sem_a, sem_b):
    _hist_body(hv_hbm, sel_hbm, out_hbm, buf, hist, red, bvec, sem_a, sem_b,
               shift=7, filt_shift=19)


def _find_bucket(h_ref, kval):
    hs = jnp.sum(h_ref[...], axis=0, keepdims=True)
    bidx = lax.broadcasted_iota(jnp.int32, (1, B), 1)

    def count_ge(t):
        return jnp.sum(jnp.where(bidx >= t, hs, 0))

    lo, hi = jnp.int32(0), jnp.int32(B)
    for _ in range(12):
        mid = (lo + hi) // 2
        good = count_ge(mid) >= kval
        lo = jnp.where(good, mid, lo)
        hi = jnp.where(good, hi, mid)
    above = count_ge(lo + 1)
    return lo, kval - above


def _tc_sel1_body(h_ref, o_ref):
    b1, krem = _find_bucket(h_ref, jnp.int32(K))
    o_ref[...] = jnp.stack([jnp.full((128,), b1, jnp.int32),
                            jnp.full((128,), krem, jnp.int32)])


def _tc_sel2_body(h_ref, sel1_ref, o_ref):
    b1 = sel1_ref[0, 0]
    krem = sel1_ref[1, 0]
    b2, _ = _find_bucket(h_ref, krem)
    t = (b1 << 19) | (b2 << 7)
    o_ref[...] = lax.bitcast_convert_type(
        jnp.stack([jnp.full((128,), t, jnp.int32),
                   jnp.full((128,), t, jnp.int32)]), jnp.float32)


FCHUNK = 16384
NFCHUNK = E // FCHUNK


@functools.partial(pl.kernel,
                   out_type=jax.ShapeDtypeStruct((N,), jnp.float32),
                   mesh=_mesh,
                   scratch_types=[
                       pltpu.VMEM((2, FCHUNK), jnp.float32),
                       pltpu.VMEM((2, FCHUNK), jnp.float32),
                       pltpu.VMEM((L,), jnp.float32),
                       pltpu.SemaphoreType.DMA,
                       pltpu.SemaphoreType.DMA,
                       pltpu.SemaphoreType.DMA,
                       pltpu.SemaphoreType.DMA,
                   ],
                   compiler_params=pltpu.CompilerParams(
                       needs_layout_passes=False))
def _sc_final(hv_hbm, sel_hbm, out_hbm, ibuf, obuf, tvec,
              isem_a, isem_b, osem_a, osem_b):
    wid = _worker_id()
    base = wid * E
    pltpu.sync_copy(sel_hbm.at[0, pl.ds(0, L)], tvec)
    tval = tvec[...]

    isems = (isem_a, isem_b)
    osems = (osem_a, osem_b)
    in_copies = [
        pltpu.make_async_copy(
            hv_hbm.at[pl.ds(base + ch * FCHUNK, FCHUNK)], ibuf.at[ch % 2],
            isems[ch % 2])
        for ch in range(NFCHUNK)
    ]
    out_copies = [
        pltpu.make_async_copy(
            obuf.at[ch % 2], out_hbm.at[pl.ds(base + ch * FCHUNK, FCHUNK)],
            osems[ch % 2])
        for ch in range(NFCHUNK)
    ]
    in_copies[0].start()
    for ch in range(NFCHUNK):
        if ch + 1 < NFCHUNK:
            in_copies[ch + 1].start()
        in_copies[ch].wait()
        if ch >= 2:
            out_copies[ch - 2].wait()
        slot = ch % 2

        @plsc.parallel_loop(0, FCHUNK // L, 1, unroll=UNROLL)
        def _(i):
            v = ibuf[slot, pl.ds(i * L, L)]
            keep = jnp.abs(v) >= tval
            obuf[slot, pl.ds(i * L, L)] = jnp.where(
                keep, jnp.sign(v), jnp.float32(0.0))
        out_copies[ch].start()
    out_copies[NFCHUNK - 2].wait()
    out_copies[NFCHUNK - 1].wait()


def kernel(hv):
    h1 = _sc_hist1(hv)
    sel1 = pl.pallas_call(
        _tc_sel1_body,
        out_shape=jax.ShapeDtypeStruct((2, 128), jnp.int32))(h1)
    h2 = _sc_hist2(hv, sel1)
    sel2 = pl.pallas_call(
        _tc_sel2_body,
        out_shape=jax.ShapeDtypeStruct((2, 128), jnp.float32))(h2, sel1)
    return _sc_final(hv, sel2)


# speedup vs baseline: 1.2145x; 1.2145x over previous
import functools

import jax
import jax.numpy as jnp
from jax import lax
from jax.experimental import pallas as pl
from jax.experimental.pallas import tpu as pltpu
from jax.experimental.pallas import tpu_sc as plsc

N = 4194304
K = N // 2
NC, NS, L = 2, 16, 16
NW = NC * NS
E = N // NW
CHUNK = 8192
NCHUNK = E // CHUNK
B = 4096
UNROLL = 8
RSTRIDE = B + 1
HSIZE = -(-(L * RSTRIDE) // (L * UNROLL)) * (L * UNROLL)

_mesh = plsc.VectorSubcoreMesh(core_axis_name="c", subcore_axis_name="s")


def _worker_id():
    return lax.axis_index("s") * NC + lax.axis_index("c")


def _hist_body(hv_hbm, sel_hbm, out_hbm, buf, hist, red, bvec, sem_a, sem_b,
               *, shift, filt_shift):
    wid = _worker_id()
    base = wid * E
    lane = lax.iota(jnp.int32, L)
    laneoff = lane * RSTRIDE
    ones = jnp.ones((L,), jnp.int32)
    zeros = jnp.zeros((L,), jnp.int32)

    if filt_shift is not None:
        pltpu.sync_copy(sel_hbm.at[0, pl.ds(0, L)], bvec)
        fval = bvec[...]

    @plsc.parallel_loop(0, HSIZE // L, 1, unroll=UNROLL)
    def _(i):
        hist[pl.ds(i * L, L)] = zeros

    sems = (sem_a, sem_b)
    copies = [
        pltpu.make_async_copy(
            hv_hbm.at[pl.ds(base + ch * CHUNK, CHUNK)], buf.at[ch % 2],
            sems[ch % 2])
        for ch in range(NCHUNK)
    ]
    copies[0].start()
    for ch in range(NCHUNK):
        if ch + 1 < NCHUNK:
            copies[ch + 1].start()
        copies[ch].wait()
        slot = ch % 2

        @plsc.parallel_loop(0, CHUNK // L, 1, unroll=UNROLL)
        def _(i):
            v = buf[slot, pl.ds(i * L, L)]
            bits = lax.bitcast_convert_type(v, jnp.int32)
            a = bits & jnp.int32(0x7FFFFFFF)
            bucket = lax.shift_right_logical(a, shift) & jnp.int32(B - 1)
            idx = laneoff + bucket
            if filt_shift is None:
                plsc.addupdate_scatter(hist, [idx], ones)
            else:
                m = lax.shift_right_logical(a, filt_shift) == fval
                plsc.addupdate_scatter(hist, [idx], ones, mask=m)

    @plsc.parallel_loop(0, B // L, 1, unroll=2)
    def _(j):
        acc = hist[pl.ds(j * L, L)]
        for l in range(1, L):
            acc = acc + hist[pl.ds(l * RSTRIDE + j * L, L)]
        red[pl.ds(j * L, L)] = acc
    pltpu.sync_copy(red, out_hbm.at[wid])


_hist_scratch = [
    pltpu.VMEM((2, CHUNK), jnp.float32),
    pltpu.VMEM((HSIZE,), jnp.int32),
    pltpu.VMEM((B,), jnp.int32),
    pltpu.VMEM((L,), jnp.int32),
    pltpu.SemaphoreType.DMA,
    pltpu.SemaphoreType.DMA,
]


@functools.partial(pl.kernel,
                   out_type=jax.ShapeDtypeStruct((NW, B), jnp.int32),
                   mesh=_mesh, scratch_types=_hist_scratch,
                   compiler_params=pltpu.CompilerParams(
                       needs_layout_passes=False))
def _sc_hist1(hv_hbm, out_hbm, buf, hist, red, bvec, sem_a, sem_b):
    _hist_body(hv_hbm, None, out_hbm, buf, hist, red, bvec, sem_a, sem_b,
               shift=19, filt_shift=None)


@functools.partial(pl.kernel,
                   out_type=jax.ShapeDtypeStruct((NW, B), jnp.int32),
                   mesh=_mesh, scratch_types=_hist_scratch,
                   compiler_params=pltpu.CompilerParams(
                       needs_layout_passes=False))
def _sc_hist2(hv_hbm, sel_hbm, out_hbm, buf, hist, red, bvec, sem_a, sem_b):
    _hist_body(hv_hbm, sel_hbm, out_hbm, buf, hist, red, bvec, sem_a, sem_b,
               shift=7, filt_shift=19)


def _find_bucket(h_ref, kval):
    hs = jnp.sum(h_ref[...], axis=0, keepdims=True)
    bidx = lax.broadcasted_iota(jnp.int32, (1, B), 1)

    def count_ge(t):
        return jnp.sum(jnp.where(bidx >= t, hs, 0))

    lo, hi = jnp.int32(0), jnp.int32(B)
    for _ in range(12):
        mid = (lo + hi) // 2
        good = count_ge(mid) >= kval
        lo = jnp.where(good, mid, lo)
        hi = jnp.where(good, hi, mid)
    above = count_ge(lo + 1)
    return lo, kval - above


def _tc_sel1_body(h_ref, o_ref):
    b1, krem = _find_bucket(h_ref, jnp.int32(K))
    o_ref[...] = jnp.stack([jnp.full((128,), b1, jnp.int32),
                            jnp.full((128,), krem, jnp.int32)])


def _tc_sel2_body(h_ref, sel1_ref, o_ref):
    b1 = sel1_ref[0, 0]
    krem = sel1_ref[1, 0]
    b2, _ = _find_bucket(h_ref, krem)
    t = (b1 << 19) | (b2 << 7)
    o_ref[...] = lax.bitcast_convert_type(
        jnp.stack([jnp.full((128,), t, jnp.int32),
                   jnp.full((128,), t, jnp.int32)]), jnp.float32)


FCHUNK = 16384
NFCHUNK = E // FCHUNK


@functools.partial(pl.kernel,
                   out_type=jax.ShapeDtypeStruct((N,), jnp.float32),
                   mesh=_mesh,
                   scratch_types=[
                       pltpu.VMEM((2, FCHUNK), jnp.float32),
                       pltpu.VMEM((2, FCHUNK), jnp.float32),
                       pltpu.VMEM((L,), jnp.float32),
                       pltpu.SemaphoreType.DMA,
                       pltpu.SemaphoreType.DMA,
                       pltpu.SemaphoreType.DMA,
                       pltpu.SemaphoreType.DMA,
                   ],
                   compiler_params=pltpu.CompilerParams(
                       needs_layout_passes=False))
def _sc_final(hv_hbm, sel_hbm, out_hbm, ibuf, obuf, tvec,
              isem_a, isem_b, osem_a, osem_b):
    wid = _worker_id()
    base = wid * E
    pltpu.sync_copy(sel_hbm.at[0, pl.ds(0, L)], tvec)
    tval = tvec[...]

    isems = (isem_a, isem_b)
    osems = (osem_a, osem_b)
    in_copies = [
        pltpu.make_async_copy(
            hv_hbm.at[pl.ds(base + ch * FCHUNK, FCHUNK)], ibuf.at[ch % 2],
            isems[ch % 2])
        for ch in range(NFCHUNK)
    ]
    out_copies = [
        pltpu.make_async_copy(
            obuf.at[ch % 2], out_hbm.at[pl.ds(base + ch * FCHUNK, FCHUNK)],
            osems[ch % 2])
        for ch in range(NFCHUNK)
    ]
    in_copies[0].start()
    for ch in range(NFCHUNK):
        if ch + 1 < NFCHUNK:
            in_copies[ch + 1].start()
        in_copies[ch].wait()
        if ch >= 2:
            out_copies[ch - 2].wait()
        slot = ch % 2

        @plsc.parallel_loop(0, FCHUNK // L, 1, unroll=UNROLL)
        def _(i):
            v = ibuf[slot, pl.ds(i * L, L)]
            keep = jnp.abs(v) >= tval
            obuf[slot, pl.ds(i * L, L)] = jnp.where(
                keep, jnp.sign(v), jnp.float32(0.0))
        out_copies[ch].start()
    out_copies[NFCHUNK - 2].wait()
    out_copies[NFCHUNK - 1].wait()


def kernel(hv):
    h1 = _sc_hist1(hv)
    sel1 = pl.pallas_call(
        _tc_sel1_body,
        out_shape=jax.ShapeDtypeStruct((2, 128), jnp.int32))(h1)
    h2 = _sc_hist2(hv, sel1)
    sel2 = pl.pallas_call(
        _tc_sel2_body,
        out_shape=jax.ShapeDtypeStruct((2, 128), jnp.float32))(h2, sel1)
    return _sc_final(hv, sel2)
